# baseline (speedup 1.0000x reference)
import functools

import numpy as np
import jax
import jax.numpy as jnp
from jax import lax
from jax.experimental import pallas as pl
from jax.experimental.pallas import tpu as pltpu
from jax.experimental.pallas import tpu_sc as plsc

LEVELS = 16
HASH_SIZE = 1 << 20
HASH_MASK = HASH_SIZE - 1
BASE = 16.0
FINEST = 2048.0
RATIO = float(np.exp((np.log(FINEST) - np.log(BASE)) / (LEVELS - 1)))
RES = np.array([int(np.floor(BASE * (RATIO ** l))) for l in range(LEVELS)],
               dtype=np.float32)
P1 = np.int32(np.uint32(2654435761).astype(np.int32))
P2 = np.int32(np.uint32(805459861).astype(np.int32))
N_POINTS = 262144
UNITS = 64

NW = 32
PW = N_POINTS // NW
C = 64
NCHUNK = PW // C
NG = C // 16


def _sc_body(pos_hbm, tab_hbm, res_hbm, enc_hbm, mask_hbm,
             pos_v, idx_buf, rem_buf, rows_v, w_buf, enc_buf, mask_buf,
             res_v, sem):
    wid = lax.axis_index("s") * 2 + lax.axis_index("c")
    pltpu.sync_copy(res_hbm, res_v)
    iota = lax.iota(jnp.int32, 16)

    def chunk_body(chunk, _):
        pbase = wid * PW + chunk * C
        pltpu.sync_copy(pos_hbm.at[pl.ds(pbase * 3, C * 3)], pos_v)

        def p1_group(g, _):
            i3 = iota * 3 + g * 48
            x = plsc.load_gather(pos_v, [i3])
            y = plsc.load_gather(pos_v, [i3 + 1])
            z = plsc.load_gather(pos_v, [i3 + 2])
            xc = jnp.minimum(jnp.maximum(x, -1.0), 1.0)
            yc = jnp.minimum(jnp.maximum(y, -1.0), 1.0)
            zc = jnp.minimum(jnp.maximum(z, -1.0), 1.0)
            inb = jnp.logical_and(jnp.logical_and(x == xc, y == yc), z == zc)
            mask_buf[pl.ds(g * 16, 16)] = jnp.where(inb, 1.0, 0.0).astype(jnp.float32)
            lx = (xc + 1.0) * 0.5
            ly = (yc + 1.0) * 0.5
            lz = (zc + 1.0) * 0.5
            ivec = iota + g * 16

            def p1_level(l, _):
                res = plsc.load_gather(res_v, [jnp.full((16,), l, jnp.int32)])
                px = lx * res
                py = ly * res
                pz = lz * res
                ix = px.astype(jnp.int32)
                iy = py.astype(jnp.int32)
                iz = pz.astype(jnp.int32)
                fx = px - ix.astype(jnp.float32)
                fy = py - iy.astype(jnp.float32)
                fz = pz - iz.astype(jnp.float32)
                hx0 = ix
                hx1 = ix + 1
                hy0 = iy * P1
                hy1 = hy0 + P1
                hz0 = iz * P2
                hz1 = hz0 + P2
                wx1, wx0 = fx, 1.0 - fx
                wy1, wy0 = fy, 1.0 - fy
                wz1, wz0 = fz, 1.0 - fz
                w00 = wx0 * wy0
                w10 = wx1 * wy0
                w01 = wx0 * wy1
                w11 = wx1 * wy1
                lbase = l * HASH_SIZE
                hx = (hx0, hx1)
                hy = (hy0, hy1)
                hz = (hz0, hz1)
                wxy = (w00, w10, w01, w11)
                wz = (wz0, wz1)
                for c in range(8):
                    bx, by, bz = c & 1, (c >> 1) & 1, (c >> 2) & 1
                    h = (hx[bx] ^ hy[by] ^ hz[bz]) & HASH_MASK
                    jv = jnp.full((16,), l * 8 + c, jnp.int32)
                    plsc.store_scatter(idx_buf, [ivec, jv],
                                       lax.shift_right_logical(h + lbase, 1))
                    plsc.store_scatter(rem_buf, [ivec, jv],
                                       lax.shift_left(h & 1, 2))
                    w_off = ((g * 16 + l) * 8 + c) * 16
                    w_buf[pl.ds(w_off, 16)] = wxy[c & 3] * wz[bz]
                return 0

            lax.fori_loop(0, LEVELS, p1_level, 0)
            return 0

        lax.fori_loop(0, NG, p1_group, 0)

        def fire(j, _):
            pltpu.async_copy(tab_hbm.at[idx_buf.at[j]], rows_v.at[j], sem)
            return 0

        def drain(j, _):
            pltpu.make_async_copy(tab_hbm.at[idx_buf.at[j]], rows_v.at[j], sem).wait()
            return 0

        lax.fori_loop(0, C, fire, 0)
        lax.fori_loop(0, C, drain, 0)

        def p2_group(g, _):
            ivec = iota + g * 16
            e64 = iota * UNITS + g * (16 * UNITS)

            def p2_level(l, _):
                acc = [jnp.zeros((16,), jnp.float32) for _ in range(4)]
                for c in range(8):
                    w_off = ((g * 16 + l) * 8 + c) * 16
                    w = w_buf[pl.ds(w_off, 16)]
                    jv = jnp.full((16,), l * 8 + c, jnp.int32)
                    rem4 = plsc.load_gather(rem_buf, [ivec, jv])
                    for f in range(4):
                        acc[f] = acc[f] + w * plsc.load_gather(
                            rows_v, [ivec, jv, rem4 + f])
                for f in range(4):
                    plsc.store_scatter(enc_buf, [e64 + (l * 4 + f)], acc[f])
                return 0

            lax.fori_loop(0, LEVELS, p2_level, 0)
            return 0

        lax.fori_loop(0, NG, p2_group, 0)

        pltpu.sync_copy(enc_buf, enc_hbm.at[pl.ds(pbase * UNITS, C * UNITS)])
        pltpu.sync_copy(mask_buf, mask_hbm.at[pl.ds(pbase, C)])
        return 0

    lax.fori_loop(0, NCHUNK, chunk_body, 0)


def _make_sc_encoder():
    mesh = plsc.VectorSubcoreMesh(core_axis_name="c", subcore_axis_name="s")
    return pl.kernel(
        _sc_body,
        mesh=mesh,
        compiler_params=pltpu.CompilerParams(needs_layout_passes=False,
                                             use_tc_tiling_on_sc=False),
        out_type=[
            jax.ShapeDtypeStruct((N_POINTS * UNITS,), jnp.float32),
            jax.ShapeDtypeStruct((N_POINTS,), jnp.float32),
        ],
        scratch_types=[
            pltpu.VMEM((C * 3,), jnp.float32),
            pltpu.VMEM((C, LEVELS * 8), jnp.int32),
            pltpu.VMEM((C, LEVELS * 8), jnp.int32),
            pltpu.VMEM((C, LEVELS * 8, 8), jnp.float32),
            pltpu.VMEM((C * LEVELS * 8,), jnp.float32),
            pltpu.VMEM((C * UNITS,), jnp.float32),
            pltpu.VMEM((C,), jnp.float32),
            pltpu.VMEM((LEVELS,), jnp.float32),
            pltpu.SemaphoreType.DMA,
        ],
    )


def _mlp_body(enc_ref, w1_ref, w2_ref, geo_ref, col_ref):
    h = jnp.maximum(jnp.dot(enc_ref[...], w1_ref[...],
                            preferred_element_type=jnp.float32), 0.0)
    o = jnp.maximum(jnp.dot(h, w2_ref[...],
                            preferred_element_type=jnp.float32), 0.0)
    geo_ref[...] = o[:, :UNITS]
    col_ref[...] = o[:, UNITS:]


_BN = 1024


def _run_mlp(enc, w1b, w2b):
    grid = (N_POINTS // _BN,)
    return pl.pallas_call(
        _mlp_body,
        grid=grid,
        in_specs=[
            pl.BlockSpec((_BN, UNITS), lambda i: (i, 0)),
            pl.BlockSpec((UNITS, 2 * UNITS), lambda i: (0, 0)),
            pl.BlockSpec((2 * UNITS, 2 * UNITS), lambda i: (0, 0)),
        ],
        out_specs=[
            pl.BlockSpec((_BN, UNITS), lambda i: (i, 0)),
            pl.BlockSpec((_BN, UNITS), lambda i: (i, 0)),
        ],
        out_shape=[
            jax.ShapeDtypeStruct((N_POINTS, UNITS), jnp.float32),
            jax.ShapeDtypeStruct((N_POINTS, UNITS), jnp.float32),
        ],
    )(enc, w1b, w2b)


def kernel(pos_xyz, geo_table, geo_W1, geo_W2, color_table, color_W1, color_W2):
    tab = jnp.concatenate(
        [geo_table.reshape(LEVELS * HASH_SIZE, 2),
         color_table.reshape(LEVELS * HASH_SIZE, 2)],
        axis=1).reshape(LEVELS * HASH_SIZE // 2, 8)
    pos_flat = pos_xyz.reshape(-1)
    res_arr = jnp.asarray(RES)

    enc_flat, mask = _make_sc_encoder()(pos_flat, tab, res_arr)
    enc = enc_flat.reshape(N_POINTS, UNITS)

    g1 = geo_W1.reshape(LEVELS, 2, UNITS)
    c1 = color_W1.reshape(LEVELS, 2, UNITS)
    w1b = jnp.zeros((LEVELS, 4, 2 * UNITS), jnp.float32)
    w1b = w1b.at[:, 0:2, :UNITS].set(g1).at[:, 2:4, UNITS:].set(c1)
    w1b = w1b.reshape(4 * LEVELS, 2 * UNITS)
    w2b = jnp.zeros((2 * UNITS, 2 * UNITS), jnp.float32)
    w2b = w2b.at[:UNITS, :UNITS].set(geo_W2).at[UNITS:, UNITS:].set(color_W2)

    geo_out, col_out = _run_mlp(enc, w1b, w2b)
    return (geo_out, col_out, mask)
